# baseline (speedup 1.0000x reference)
import jax
import jax.numpy as jnp
from jax.experimental import pallas as pl
from jax.experimental.pallas import tpu as pltpu

N_NODES = 10000
HEADS = 2
MSG_DIM = 128


def _matmul_body(m_ref, w_ref, o_ref):
    o_ref[...] = jnp.dot(m_ref[...], w_ref[...],
                         preferred_element_type=jnp.float32)


def _logits(messages, W):
    E = messages.shape[0]
    BLK = 2000
    return pl.pallas_call(
        _matmul_body,
        grid=(E // BLK,),
        in_specs=[
            pl.BlockSpec((BLK, MSG_DIM), lambda i: (i, 0)),
            pl.BlockSpec((MSG_DIM, MSG_DIM * HEADS), lambda i: (0, 0)),
        ],
        out_specs=pl.BlockSpec((BLK, MSG_DIM * HEADS), lambda i: (i, 0)),
        out_shape=jax.ShapeDtypeStruct((E, MSG_DIM * HEADS), jnp.float32),
    )(messages, W)


def kernel(messages, edge_index, W):
    E = messages.shape[0]
    logits = _logits(messages, W).reshape(E, HEADS, MSG_DIM)
    src = edge_index[0]
    seg_max = jax.ops.segment_max(logits, src, num_segments=N_NODES)
    w = jnp.exp(logits - seg_max[src])
    seg_sum = jax.ops.segment_sum(w, src, num_segments=N_NODES)
    return w / seg_sum[src]


# speedup vs baseline: 9.7036x; 9.7036x over previous
import functools

import jax
import jax.numpy as jnp
from jax import lax
from jax.experimental import pallas as pl
from jax.experimental.pallas import tpu as pltpu
from jax.experimental.pallas import tpu_sc as plsc

N_NODES = 10000
HEADS = 2
D = 128
C = HEADS * D
E = 320000

NC, NS = 2, 16
NW = NC * NS
NROUND = 2
NPART = NW * NROUND
NPN = 157
CH = 64
SCAN = 3200
EID_BITS = 19
EID_MASK = (1 << EID_BITS) - 1
E_PAD = E + CH
PAGE = 96
NEG = -3.0e38


def _matmul_body(m_ref, w_ref, o_ref):
    o_ref[...] = jnp.dot(m_ref[...], w_ref[...],
                         preferred_element_type=jnp.float32)


def _logits(messages, W):
    BLK = 2000
    return pl.pallas_call(
        _matmul_body,
        grid=(E // BLK,),
        in_specs=[
            pl.BlockSpec((BLK, D), lambda i: (i, 0)),
            pl.BlockSpec((D, C), lambda i: (0, 0)),
        ],
        out_specs=pl.BlockSpec((BLK, C), lambda i: (i, 0)),
        out_shape=jax.ShapeDtypeStruct((E, C), jnp.float32),
    )(messages, W)


def _sc_body(src_hbm, logits_hbm, out_hbm,
             bins_hbm, scanbuf, page0, page1, pbuf, eid_buf, xbuf, obuf,
             mtab, stab, sem_scan, sem_g, sem_s):
    c = lax.axis_index("c")
    s = lax.axis_index("s")
    w = s * NC + c
    lane = lax.iota(jnp.int32, 16)

    lo0 = w * NPN
    lo1 = (NW + w) * NPN

    def scan_blk(b, carry):
        pltpu.sync_copy(src_hbm.at[pl.ds(b * SCAN, SCAN)], scanbuf)

        def scan_chunk(i, carry):
            f0, wo0, f1, wo1 = carry
            srcv = scanbuf[pl.ds(i * 16, 16)]
            eidv = (b * SCAN + i * 16) + lane

            def one_part(page, lo, part, f, wo):
                node = srcv - lo
                msk = (node >= 0) & (node < NPN)
                packed = (node << EID_BITS) | eidv
                plsc.store_compressed(page.at[pl.ds(f, 16)], packed, mask=msk)
                f = f + jnp.sum(msk.astype(jnp.int32))

                def flush():
                    pltpu.sync_copy(page.at[pl.ds(0, CH)],
                                    bins_hbm.at[pl.ds(pl.multiple_of(part * E_PAD + wo, CH), CH)])
                    page[pl.ds(0, 16)] = page[pl.ds(CH, 16)]
                pl.when(f >= CH)(flush)
                full = (f >= CH).astype(jnp.int32)
                return f - full * CH, wo + full * CH

            f0, wo0 = one_part(page0, lo0, w, f0, wo0)
            f1, wo1 = one_part(page1, lo1, NW + w, f1, wo1)
            return f0, wo0, f1, wo1

        return lax.fori_loop(0, SCAN // 16, scan_chunk, carry)

    f0, wo0, f1, wo1 = lax.fori_loop(0, E // SCAN, scan_blk, (0, 0, 0, 0))
    pltpu.sync_copy(page0.at[pl.ds(0, CH)], bins_hbm.at[pl.ds(pl.multiple_of(w * E_PAD + wo0, CH), CH)])
    pltpu.sync_copy(page1.at[pl.ds(0, CH)],
                    bins_hbm.at[pl.ds(pl.multiple_of((NW + w) * E_PAD + wo1, CH), CH)])
    cnts = (wo0 + f0, wo1 + f1)

    for r in range(NROUND):
        p = r * NW + w
        cnt = cnts[r]

        def init_node(i, _):
            for j in range(C // 16):
                mtab[pl.ds(i * C + j * 16, 16)] = jnp.full((16,), NEG,
                                                           jnp.float32)
                stab[pl.ds(i * C + j * 16, 16)] = jnp.zeros((16,),
                                                            jnp.float32)
            return _
        lax.fori_loop(0, NPN, init_node, None)

        nchunks = (cnt + CH - 1) // CH

        def stage_chunk(ci):
            off = ci * CH
            pltpu.sync_copy(bins_hbm.at[pl.ds(pl.multiple_of(p * E_PAD + off, CH), CH)],
                            pbuf.at[pl.ds(0, CH)])
            nrem = cnt - off
            for q in range(CH // 16):
                pk = pbuf[pl.ds(q * 16, 16)]
                eidv = pk & EID_MASK
                valid = (q * 16 + lane) < nrem
                eidv = jnp.where(valid, eidv, E + q * 16 + lane)
                eid_buf[pl.ds(q * 16, 16)] = eidv
            return nrem

        def chunk_a(ci, _):
            nrem = stage_chunk(ci)
            pltpu.async_copy(logits_hbm.at[eid_buf], xbuf, sem_g).wait()

            def edge_a(e, _):
                pk = pbuf[pl.ds(e, 16)][0]
                base = (pk >> EID_BITS) * C
                for j in range(C // 16):
                    x = xbuf[e, pl.ds(j * 16, 16)]
                    mo = mtab[pl.ds(base + j * 16, 16)]
                    so = stab[pl.ds(base + j * 16, 16)]
                    mn = jnp.maximum(mo, x)
                    sn = so * jnp.exp(mo - mn) + jnp.exp(x - mn)
                    mtab[pl.ds(base + j * 16, 16)] = mn
                    stab[pl.ds(base + j * 16, 16)] = sn
                return _
            lax.fori_loop(0, jnp.minimum(nrem, CH), edge_a, None)
            return _
        lax.fori_loop(0, nchunks, chunk_a, None)

        def chunk_b(ci, _):
            nrem = stage_chunk(ci)
            pltpu.async_copy(logits_hbm.at[eid_buf], xbuf, sem_g).wait()

            def edge_b(e, _):
                pk = pbuf[pl.ds(e, 16)][0]
                base = (pk >> EID_BITS) * C
                for j in range(C // 16):
                    x = xbuf[e, pl.ds(j * 16, 16)]
                    m = mtab[pl.ds(base + j * 16, 16)]
                    sm = stab[pl.ds(base + j * 16, 16)]
                    obuf[e, pl.ds(j * 16, 16)] = jnp.exp(x - m) / sm
                return _
            lax.fori_loop(0, jnp.minimum(nrem, CH), edge_b, None)
            pltpu.async_copy(obuf, out_hbm.at[eid_buf], sem_s).wait()
            return _
        lax.fori_loop(0, nchunks, chunk_b, None)


def _edge_softmax(src, logits):
    mesh = plsc.VectorSubcoreMesh(core_axis_name="c", subcore_axis_name="s",
                                  num_cores=NC, num_subcores=NS)
    f = pl.kernel(
        _sc_body,
        out_type=jax.ShapeDtypeStruct((E + CH, C), jnp.float32),
        mesh=mesh,
        compiler_params=pltpu.CompilerParams(needs_layout_passes=False),
        scratch_types=[
            pltpu.HBM((NPART * E_PAD,), jnp.int32),
            pltpu.VMEM((SCAN,), jnp.int32),
            pltpu.VMEM((PAGE,), jnp.int32),
            pltpu.VMEM((PAGE,), jnp.int32),
            pltpu.VMEM((CH + 16,), jnp.int32),
            pltpu.VMEM((CH,), jnp.int32),
            pltpu.VMEM((CH, C), jnp.float32),
            pltpu.VMEM((CH, C), jnp.float32),
            pltpu.VMEM((NPN * C,), jnp.float32),
            pltpu.VMEM((NPN * C,), jnp.float32),
            pltpu.SemaphoreType.DMA,
            pltpu.SemaphoreType.DMA,
            pltpu.SemaphoreType.DMA,
        ],
    )
    return f(src, logits)


def kernel(messages, edge_index, W):
    logits = _logits(messages, W)
    src = edge_index[0].astype(jnp.int32)
    out = _edge_softmax(src, logits)
    return out[:E].reshape(E, HEADS, D)
